# Initial kernel scaffold; baseline (speedup 1.0000x reference)
import jax
import jax.numpy as jnp
from jax.experimental import pallas as pl


def kernel(x):
    raise NotImplementedError("write your pallas kernel here")


# speedup vs baseline: 2.0037x; 2.0037x over previous
import functools

import jax
import jax.numpy as jnp
from jax import lax
from jax.experimental import pallas as pl
from jax.experimental.pallas import tpu as pltpu

LANE = 128
LANE_BITS = 7


def _bit(iota_lane, iota_sub, t):
    if t < LANE_BITS:
        return ((iota_lane >> t) & 1).astype(jnp.bool_)
    return ((iota_sub >> (t - LANE_BITS)) & 1).astype(jnp.bool_)


def _roll(x, shift, axis):
    return pltpu.roll(x, shift % x.shape[axis], axis)


def _sort_body(n_bits, x_ref, o_ref):
    x = x_ref[...]
    shape = x.shape
    iota_lane = lax.broadcasted_iota(jnp.int32, shape, 2)
    iota_sub = lax.broadcasted_iota(jnp.int32, shape, 1)

    for k in range(1, n_bits + 1):
        if k < n_bits:
            is_desc = jnp.logical_not(_bit(iota_lane, iota_sub, k))
        else:
            is_desc = None
        for j in range(k - 1, -1, -1):
            s = 1 << j
            if j < LANE_BITS:
                axis = 2
                sh = s
            else:
                axis = 1
                sh = s >> LANE_BITS
            r_lo = _roll(x, sh, axis)
            r_hi = _roll(x, -sh, axis)
            bit_j = _bit(iota_lane, iota_sub, j)
            partner = jnp.where(bit_j, r_lo, r_hi)
            mn = jnp.minimum(x, partner)
            mx = jnp.maximum(x, partner)
            if is_desc is None:
                take_min = bit_j
            else:
                take_min = bit_j == is_desc
            x = jnp.where(take_min, mn, mx)
    o_ref[...] = x


def _make_sort(rows, cols, block_rows, interpret=False):
    assert cols % LANE == 0
    sub = cols // LANE
    n_bits = (cols - 1).bit_length()
    assert 1 << n_bits == cols
    grid = rows // block_rows

    return pl.pallas_call(
        functools.partial(_sort_body, n_bits),
        grid=(grid,),
        in_specs=[pl.BlockSpec((block_rows, sub, LANE), lambda i: (i, 0, 0))],
        out_specs=pl.BlockSpec((block_rows, sub, LANE), lambda i: (i, 0, 0)),
        out_shape=jax.ShapeDtypeStruct((rows, sub, LANE), jnp.float32),
        interpret=interpret,
    )


@jax.jit
def kernel(x):
    rows, cols = x.shape
    x3 = x.reshape(rows, cols // LANE, LANE)
    out = _make_sort(rows, cols, block_rows=8)(x3)
    return out.reshape(rows, cols)


# speedup vs baseline: 3.2369x; 1.6154x over previous
import functools

import jax
import jax.numpy as jnp
from jax import lax
from jax.experimental import pallas as pl
from jax.experimental.pallas import tpu as pltpu
from jax.experimental.pallas import tpu_sc as plsc

ROWS = 128
N = 32768
L = 16
NV = N // L
NC = 2
NS = 16
NW = NC * NS
RPW = ROWS // NW

_PASSES = ((0, 11), (11, 11), (22, 10))
_HOFF = (0, 2048, 4096)
_HIST = 2048 + 2048 + 1024


def _desc_key(v):
    return jnp.where(v >= 0, v ^ 0x7FFFFFFF, v)


def _digit(k, sh, nb, ho):
    d = lax.shift_right_logical(k, sh) if sh else k
    return jnp.bitwise_and(d, (1 << nb) - 1) + ho


def _sc_body(x_hbm, out_hbm, buf_a, buf_b, hist):
    cid = lax.axis_index("c")
    sid = lax.axis_index("s")
    wid = sid * NC + cid

    zeros = jnp.zeros((L,), jnp.int32)

    for r in range(RPW):
        row = wid * RPW + r
        pltpu.sync_copy(x_hbm.at[row], buf_a)

        def zf(i, _):
            hist[pl.ds(i * L, L)] = zeros
            return 0
        lax.fori_loop(0, _HIST // L, zf, 0)

        def h0(i, _):
            v = buf_a[pl.ds(i * L, L)]
            k = _desc_key(v)
            buf_a[pl.ds(i * L, L)] = k
            for (sh, nb), ho in zip(_PASSES, _HOFF):
                d = _digit(k, sh, nb, ho)
                cnt, last = plsc.scan_count(d)
                plsc.addupdate_scatter(hist, [d], cnt, mask=last)
            return 0
        lax.fori_loop(0, NV, h0, 0, unroll=2)

        bufs = ((buf_a, buf_b), (buf_b, buf_a), (buf_a, buf_b))
        for p, ((sh, nb), ho) in enumerate(zip(_PASSES, _HOFF)):
            src, dst = bufs[p]

            def pf(i, carry):
                v = hist[pl.ds(ho + i * L, L)]
                s = plsc.cumsum(v)
                hist[pl.ds(ho + i * L, L)] = s - v + carry
                return carry + jnp.sum(v)
            lax.fori_loop(0, (1 << nb) // L, pf, jnp.int32(0))

            last_pass = p == len(_PASSES) - 1

            def pm(i, _):
                k = src[pl.ds(i * L, L)]
                d = _digit(k, sh, nb, ho)
                cnt, lastm = plsc.scan_count(d)
                base = plsc.load_gather(hist, [d])
                pos = base + cnt - 1
                val = _desc_key(k) if last_pass else k
                plsc.store_scatter(dst, [pos], val)
                plsc.addupdate_scatter(hist, [d], cnt, mask=lastm)
                return 0
            lax.fori_loop(0, NV, pm, 0, unroll=2)

        pltpu.sync_copy(buf_b, out_hbm.at[row])


@jax.jit
def kernel(x):
    xb = lax.bitcast_convert_type(x, jnp.int32)
    mesh = plsc.VectorSubcoreMesh(core_axis_name="c", subcore_axis_name="s")
    f = pl.kernel(
        _sc_body,
        out_type=jax.ShapeDtypeStruct((ROWS, N), jnp.int32),
        mesh=mesh,
        compiler_params=pltpu.CompilerParams(needs_layout_passes=False),
        scratch_types=[
            pltpu.VMEM((N,), jnp.int32),
            pltpu.VMEM((N,), jnp.int32),
            pltpu.VMEM((_HIST,), jnp.int32),
        ],
    )
    return lax.bitcast_convert_type(f(xb), jnp.float32)


# speedup vs baseline: 3.2468x; 1.0031x over previous
import functools

import jax
import jax.numpy as jnp
from jax import lax
from jax.experimental import pallas as pl
from jax.experimental.pallas import tpu as pltpu
from jax.experimental.pallas import tpu_sc as plsc

ROWS = 128
N = 32768
L = 16
NV = N // L
NC = 2
NS = 16
NW = NC * NS
RPW = ROWS // NW

_PASSES = ((0, 11), (11, 11), (22, 10))
_HOFF = (0, 2048, 4096)
_HIST = 2048 + 2048 + 1024


def _desc_key(v):
    return jnp.where(v >= 0, v ^ 0x7FFFFFFF, v)


def _digit(k, sh, nb, ho):
    d = lax.shift_right_logical(k, sh) if sh else k
    return jnp.bitwise_and(d, (1 << nb) - 1) + ho


def _sc_body(x_hbm, out_hbm, buf_a, buf_b, hist):
    cid = lax.axis_index("c")
    sid = lax.axis_index("s")
    wid = sid * NC + cid

    zeros = jnp.zeros((L,), jnp.int32)

    for r in range(RPW):
        row = wid * RPW + r
        pltpu.sync_copy(x_hbm.at[row], buf_a)

        def zf(i, _):
            hist[pl.ds(i * L, L)] = zeros
            return 0
        lax.fori_loop(0, _HIST // L, zf, 0)

        def h0(i, _):
            v = buf_a[pl.ds(i * L, L)]
            k = _desc_key(v)
            buf_a[pl.ds(i * L, L)] = k
            for (sh, nb), ho in zip(_PASSES, _HOFF):
                d = _digit(k, sh, nb, ho)
                cnt, last = plsc.scan_count(d)
                plsc.addupdate_scatter(hist, [d], cnt, mask=last)
            return 0
        lax.fori_loop(0, NV, h0, 0, unroll=4)

        bufs = ((buf_a, buf_b), (buf_b, buf_a), (buf_a, buf_b))
        for p, ((sh, nb), ho) in enumerate(zip(_PASSES, _HOFF)):
            src, dst = bufs[p]

            def pf(i, carry):
                v = hist[pl.ds(ho + i * L, L)]
                s = plsc.cumsum(v)
                hist[pl.ds(ho + i * L, L)] = s - v + carry
                return carry + jnp.sum(v)
            lax.fori_loop(0, (1 << nb) // L, pf, jnp.int32(0))

            last_pass = p == len(_PASSES) - 1

            def pm(i, _):
                k = src[pl.ds(i * L, L)]
                d = _digit(k, sh, nb, ho)
                cnt, lastm = plsc.scan_count(d)
                base = plsc.load_gather(hist, [d])
                pos = base + cnt - 1
                val = _desc_key(k) if last_pass else k
                plsc.store_scatter(dst, [pos], val)
                plsc.addupdate_scatter(hist, [d], cnt, mask=lastm)
                return 0
            lax.fori_loop(0, NV, pm, 0, unroll=4)

        pltpu.sync_copy(buf_b, out_hbm.at[row])


@jax.jit
def kernel(x):
    xb = lax.bitcast_convert_type(x, jnp.int32)
    mesh = plsc.VectorSubcoreMesh(core_axis_name="c", subcore_axis_name="s")
    f = pl.kernel(
        _sc_body,
        out_type=jax.ShapeDtypeStruct((ROWS, N), jnp.int32),
        mesh=mesh,
        compiler_params=pltpu.CompilerParams(needs_layout_passes=False),
        scratch_types=[
            pltpu.VMEM((N,), jnp.int32),
            pltpu.VMEM((N,), jnp.int32),
            pltpu.VMEM((_HIST,), jnp.int32),
        ],
    )
    return lax.bitcast_convert_type(f(xb), jnp.float32)
